# baseline (speedup 1.0000x reference)
import functools

import jax
import jax.numpy as jnp
from jax import lax
from jax.experimental import pallas as pl
from jax.experimental.pallas import tpu as pltpu
from jax.experimental.pallas import tpu_sc as plsc

_NC = 2
_NS = 16
_NW = _NC * _NS
_B = 16384
_BPW = _B // _NW
_CW = 128
_KC = _BPW // _CW
_L = 16


def _make_sc_kernel():
    mesh = plsc.VectorSubcoreMesh(core_axis_name="c", subcore_axis_name="s")

    @functools.partial(
        pl.kernel,
        mesh=mesh,
        out_type=jax.ShapeDtypeStruct((_NW, 2, _L), jnp.float32),
        scratch_types=[
            pltpu.VMEM((_BPW,), jnp.int32),
            pltpu.VMEM((_BPW,), jnp.int32),
            pltpu.VMEM((_BPW,), jnp.float32),
            pltpu.VMEM((_BPW,), jnp.float32),
            pltpu.VMEM((_BPW,), jnp.float32),
            pltpu.VMEM((2, _L), jnp.float32),
            pltpu.SemaphoreType.DMA,
            pltpu.SemaphoreType.DMA,
        ],
    )
    def _k(su_hbm, si_hbm, r_hbm, bu_hbm, bi_hbm, out_hbm,
           idxu_v, idxi_v, bu_v, bi_v, r_v, acc_v, sem_in, sem_g):
        cid = lax.axis_index("c")
        sid = lax.axis_index("s")
        wid = sid * _NC + cid

        ld_u = pltpu.async_copy(su_hbm.at[wid], idxu_v, sem_in)
        ld_i = pltpu.async_copy(si_hbm.at[wid], idxi_v, sem_in)
        ld_r = pltpu.async_copy(r_hbm.at[wid], r_v, sem_in)
        ld_u.wait()
        ld_i.wait()
        copies = []
        for k in range(_KC):
            sl = pl.ds(k * _CW, _CW)
            copies.append(
                pltpu.async_copy(bu_hbm.at[idxu_v.at[sl]], bu_v.at[sl], sem_g))
            copies.append(
                pltpu.async_copy(bi_hbm.at[idxi_v.at[sl]], bi_v.at[sl], sem_g))
        ld_r.wait()
        for c in copies:
            c.wait()

        acc1 = jnp.zeros((_L,), jnp.float32)
        acc2 = jnp.zeros((_L,), jnp.float32)
        for t in range(_BPW // _L):
            sl = pl.ds(t * _L, _L)
            d = bu_v[sl] + bi_v[sl] - r_v[sl]
            acc1 = acc1 + d
            acc2 = acc2 + d * d
        acc_v[0, :] = acc2
        acc_v[1, :] = acc1
        pltpu.sync_copy(acc_v, out_hbm.at[wid])

    return _k


_sc_kernel = _make_sc_kernel()


def kernel(sampleU, sampleI, sampleR, alpha, betaU, betaI):
    su = sampleU.astype(jnp.int32).reshape(_NW, _BPW)
    si = sampleI.astype(jnp.int32).reshape(_NW, _BPW)
    r = sampleR.astype(jnp.float32).reshape(_NW, _BPW)
    partials = _sc_kernel(su, si, r, betaU, betaI)
    s = jnp.sum(partials, axis=(0, 2))
    a = alpha.astype(jnp.float32)
    return (s[0] + 2.0 * a * s[1] + _B * a * a) * (0.5 / _B)


# speedup vs baseline: 1.0766x; 1.0766x over previous
import functools

import jax
import jax.numpy as jnp
from jax import lax
from jax.experimental import pallas as pl
from jax.experimental.pallas import tpu as pltpu
from jax.experimental.pallas import tpu_sc as plsc

_NC = 2
_NS = 16
_NW = _NC * _NS
_B = 16384
_BPW = _B // _NW
_CW = 128
_KC = _BPW // _CW
_L = 16
_VPC = _CW // _L


def _make_sc_kernel():
    mesh = plsc.VectorSubcoreMesh(core_axis_name="c", subcore_axis_name="s")

    @functools.partial(
        pl.kernel,
        mesh=mesh,
        out_type=jax.ShapeDtypeStruct((_NW, 2, _L), jnp.float32),
        scratch_types=[
            pltpu.VMEM((_KC, _CW), jnp.int32),
            pltpu.VMEM((_KC, _CW), jnp.int32),
            pltpu.VMEM((_KC, _CW), jnp.float32),
            pltpu.VMEM((_KC, _CW), jnp.float32),
            pltpu.VMEM((_KC, _CW), jnp.float32),
            pltpu.VMEM((2, _L), jnp.float32),
            pltpu.SemaphoreType.DMA,
            pltpu.SemaphoreType.DMA,
        ],
    )
    def _k(su_hbm, si_hbm, r_hbm, bu_hbm, bi_hbm, out_hbm,
           idxu_v, idxi_v, bu_v, bi_v, r_v, acc_v, sem_in, sem_g):
        cid = lax.axis_index("c")
        sid = lax.axis_index("s")
        wid = sid * _NC + cid

        ld_u = pltpu.async_copy(su_hbm.at[wid], idxu_v, sem_in)
        ld_i = pltpu.async_copy(si_hbm.at[wid], idxi_v, sem_in)
        ld_r = pltpu.async_copy(r_hbm.at[wid], r_v, sem_in)
        ld_u.wait()
        ld_i.wait()
        copies = []
        for k in range(_KC):
            copies.append(
                pltpu.async_copy(bu_hbm.at[idxu_v.at[k]], bu_v.at[k], sem_g))
            copies.append(
                pltpu.async_copy(bi_hbm.at[idxi_v.at[k]], bi_v.at[k], sem_g))
        ld_r.wait()
        for c in copies:
            c.wait()

        acc1 = jnp.zeros((_L,), jnp.float32)
        acc2 = jnp.zeros((_L,), jnp.float32)
        for k in range(_KC):
            for j in range(_VPC):
                sl = pl.ds(j * _L, _L)
                d = bu_v[k, sl] + bi_v[k, sl] - r_v[k, sl]
                acc1 = acc1 + d
                acc2 = acc2 + d * d
        acc_v[0, :] = acc2
        acc_v[1, :] = acc1
        pltpu.sync_copy(acc_v, out_hbm.at[wid])

    return _k


_sc_kernel = _make_sc_kernel()


def kernel(sampleU, sampleI, sampleR, alpha, betaU, betaI):
    su = sampleU.astype(jnp.int32).reshape(_NW, _KC, _CW)
    si = sampleI.astype(jnp.int32).reshape(_NW, _KC, _CW)
    r = sampleR.astype(jnp.float32).reshape(_NW, _KC, _CW)
    partials = _sc_kernel(su, si, r, betaU, betaI)
    s = jnp.sum(partials, axis=(0, 2))
    a = alpha.astype(jnp.float32)
    return (s[0] + 2.0 * a * s[1] + _B * a * a) * (0.5 / _B)


# speedup vs baseline: 1.1563x; 1.0741x over previous
import functools

import jax
import jax.numpy as jnp
from jax import lax
from jax.experimental import pallas as pl
from jax.experimental.pallas import tpu as pltpu
from jax.experimental.pallas import tpu_sc as plsc

_NC = 2
_NS = 16
_NW = _NC * _NS
_B = 16384
_BPW = _B // _NW
_CW = 128
_KC = _BPW // _CW
_L = 16
_VPC = _CW // _L


def _make_sc_kernel():
    mesh = plsc.VectorSubcoreMesh(core_axis_name="c", subcore_axis_name="s")

    @functools.partial(
        pl.kernel,
        mesh=mesh,
        out_type=jax.ShapeDtypeStruct((_NW * _L,), jnp.float32),
        scratch_types=[
            pltpu.VMEM((_KC, _CW), jnp.int32),
            pltpu.VMEM((_KC, _CW), jnp.int32),
            pltpu.VMEM((_KC, _CW), jnp.float32),
            pltpu.VMEM((_KC, _CW), jnp.float32),
            pltpu.VMEM((_KC, _CW), jnp.float32),
            pltpu.VMEM((_L,), jnp.float32),
            pltpu.VMEM((_L,), jnp.float32),
            pltpu.SemaphoreType.DMA,
            pltpu.SemaphoreType.DMA,
        ],
    )
    def _k(su_hbm, si_hbm, r_hbm, alpha_hbm, bu_hbm, bi_hbm, out_hbm,
           idxu_v, idxi_v, bu_v, bi_v, r_v, alpha_v, acc_v, sem_in, sem_g):
        cid = lax.axis_index("c")
        sid = lax.axis_index("s")
        wid = sid * _NC + cid

        ld_u = pltpu.async_copy(su_hbm.at[wid], idxu_v, sem_in)
        ld_i = pltpu.async_copy(si_hbm.at[wid], idxi_v, sem_in)
        ld_r = pltpu.async_copy(r_hbm.at[wid], r_v, sem_in)
        ld_a = pltpu.async_copy(alpha_hbm, alpha_v, sem_in)
        ld_u.wait()
        ld_i.wait()
        copies = []
        for k in range(_KC):
            copies.append(
                pltpu.async_copy(bu_hbm.at[idxu_v.at[k]], bu_v.at[k], sem_g))
            copies.append(
                pltpu.async_copy(bi_hbm.at[idxi_v.at[k]], bi_v.at[k], sem_g))
        ld_r.wait()
        ld_a.wait()
        for c in copies:
            c.wait()

        av = alpha_v[...]
        acc = jnp.zeros((_L,), jnp.float32)
        for k in range(_KC):
            for j in range(_VPC):
                sl = pl.ds(j * _L, _L)
                d = av + bu_v[k, sl] + bi_v[k, sl] - r_v[k, sl]
                acc = acc + d * d
        acc_v[...] = acc * (0.5 / _B)
        pltpu.sync_copy(acc_v, out_hbm.at[pl.ds(wid * _L, _L)])

    return _k


_sc_kernel = _make_sc_kernel()


def kernel(sampleU, sampleI, sampleR, alpha, betaU, betaI):
    su = sampleU.astype(jnp.int32).reshape(_NW, _KC, _CW)
    si = sampleI.astype(jnp.int32).reshape(_NW, _KC, _CW)
    r = sampleR.astype(jnp.float32).reshape(_NW, _KC, _CW)
    al = jnp.broadcast_to(alpha.astype(jnp.float32), (_L,))
    partials = _sc_kernel(su, si, r, al, betaU, betaI)
    return jnp.sum(partials)
